# baseline (speedup 1.0000x reference)
import jax
import jax.numpy as jnp
from jax.experimental import pallas as pl

_ROW_BLOCK = 1000


def _fused_heads_kernel(x_ref, w_ref, b_ref, o_ref):
    o_ref[...] = (
        jnp.dot(x_ref[...], w_ref[...], preferred_element_type=jnp.float32,
                precision=jax.lax.Precision.HIGHEST)
        + b_ref[...]
    )


def kernel(x, W_cls, b_cls, W_box, b_box):
    if x.ndim > 2:
        x = x.reshape(x.shape[0], -1)
    R, D = x.shape
    n_cls = W_cls.shape[1]
    n_all = n_cls + W_box.shape[1]
    cp = max(128, ((n_all + 127) // 128) * 128)

    W = jnp.concatenate([W_cls, W_box], axis=1)
    W = jnp.pad(W, ((0, 0), (0, cp - n_all)))
    b = jnp.pad(jnp.concatenate([b_cls, b_box]), (0, cp - n_all)).reshape(1, cp)

    out = pl.pallas_call(
        _fused_heads_kernel,
        grid=(pl.cdiv(R, _ROW_BLOCK),),
        in_specs=[
            pl.BlockSpec((_ROW_BLOCK, D), lambda i: (i, 0)),
            pl.BlockSpec((D, cp), lambda i: (0, 0)),
            pl.BlockSpec((1, cp), lambda i: (0, 0)),
        ],
        out_specs=pl.BlockSpec((_ROW_BLOCK, cp), lambda i: (i, 0)),
        out_shape=jax.ShapeDtypeStruct((R, cp), jnp.float32),
    )(x, W, b)

    return out[:, :n_cls], out[:, n_cls:n_all]


# speedup vs baseline: 2.1585x; 2.1585x over previous
import functools

import jax
import jax.numpy as jnp
from jax.experimental import pallas as pl

_ROW_BLOCK = 1000
_NSPLIT = 4


def _fused_heads_kernel(*refs):
    xs = refs[:_NSPLIT]
    w_ref, b_ref, o_ref = refs[_NSPLIT:]
    ds = xs[0].shape[1]
    acc = b_ref[...] + jnp.dot(xs[0][...], w_ref[0:ds, :],
                               preferred_element_type=jnp.float32)
    for s in range(1, _NSPLIT):
        acc = acc + jnp.dot(xs[s][...], w_ref[s * ds:(s + 1) * ds, :],
                            preferred_element_type=jnp.float32)
    o_ref[...] = acc


def kernel(x, W_cls, b_cls, W_box, b_box):
    if x.ndim > 2:
        x = x.reshape(x.shape[0], -1)
    R, D = x.shape
    n_cls = W_cls.shape[1]
    n_all = n_cls + W_box.shape[1]
    cp = max(128, ((n_all + 127) // 128) * 128)

    W = jnp.concatenate([W_cls, W_box], axis=1)
    W = jnp.pad(W, ((0, 0), (0, cp - n_all)))
    b = jnp.pad(jnp.concatenate([b_cls, b_box]), (0, cp - n_all)).reshape(1, cp)

    ds = D // _NSPLIT
    in_specs = [
        pl.BlockSpec((_ROW_BLOCK, ds), functools.partial(lambda s, i: (i, s), s))
        for s in range(_NSPLIT)
    ]
    in_specs += [
        pl.BlockSpec((D, cp), lambda i: (0, 0)),
        pl.BlockSpec((1, cp), lambda i: (0, 0)),
    ]

    out = pl.pallas_call(
        _fused_heads_kernel,
        grid=(pl.cdiv(R, _ROW_BLOCK),),
        in_specs=in_specs,
        out_specs=pl.BlockSpec((_ROW_BLOCK, cp), lambda i: (i, 0)),
        out_shape=jax.ShapeDtypeStruct((R, cp), jnp.float32),
    )(*([x] * _NSPLIT + [W, b]))

    return out[:, :n_cls], out[:, n_cls:n_all]
